# Initial kernel scaffold; baseline (speedup 1.0000x reference)
import jax
import jax.numpy as jnp
from jax.experimental import pallas as pl


def kernel(x, W1, b1, W2, b2):
    raise NotImplementedError("write your pallas kernel here")


# speedup vs baseline: 4.9760x; 4.9760x over previous
import functools

import jax
import jax.numpy as jnp
from jax.experimental import pallas as pl

K = 8


def _gating_block_kernel(x_ref, w1_ref, b1_ref, w2_ref, b2_ref,
                         gate_ref, idx_ref):
    h = jnp.dot(x_ref[...], w1_ref[...], preferred_element_type=jnp.float32)
    h = jnp.maximum(h + b1_ref[...], 0.0)
    logits = jnp.dot(h, w2_ref[...], preferred_element_type=jnp.float32)
    logits = logits + b2_ref[...]

    r, e = logits.shape
    iota = jax.lax.broadcasted_iota(jnp.int32, (r, e), 1)
    neg_inf = jnp.float32(-jnp.inf)

    work = logits
    selected = jnp.zeros((r, e), dtype=jnp.bool_)
    idx_cols = []
    top1 = None
    for k in range(K):
        m = jnp.max(work, axis=-1, keepdims=True)
        if k == 0:
            top1 = m
        is_max = work == m
        idx = jnp.min(jnp.where(is_max, iota, e), axis=-1)
        idx_cols.append(idx)
        one_hot = iota == idx[:, None]
        selected = jnp.logical_or(selected, one_hot)
        work = jnp.where(one_hot, neg_inf, work)

    p = jnp.where(selected, jnp.exp(logits - top1), 0.0)
    z = jnp.sum(p, axis=-1, keepdims=True)
    gate_ref[...] = p / z
    idx_ref[...] = jnp.stack(idx_cols, axis=-1)


@functools.partial(jax.jit, static_argnames=("block_rows",))
def _gating(x, W1, b1, W2, b2, block_rows=512):
    n, d = x.shape
    h_dim = W1.shape[1]
    e = W2.shape[1]
    grid = (n // block_rows,)
    gate, idx = pl.pallas_call(
        _gating_block_kernel,
        grid=grid,
        in_specs=[
            pl.BlockSpec((block_rows, d), lambda i: (i, 0)),
            pl.BlockSpec((d, h_dim), lambda i: (0, 0)),
            pl.BlockSpec((1, h_dim), lambda i: (0, 0)),
            pl.BlockSpec((h_dim, e), lambda i: (0, 0)),
            pl.BlockSpec((1, e), lambda i: (0, 0)),
        ],
        out_specs=[
            pl.BlockSpec((block_rows, e), lambda i: (i, 0)),
            pl.BlockSpec((block_rows, K), lambda i: (i, 0)),
        ],
        out_shape=[
            jax.ShapeDtypeStruct((n, e), jnp.float32),
            jax.ShapeDtypeStruct((n, K), jnp.int32),
        ],
    )(x, W1, b1.reshape(1, -1), W2, b2.reshape(1, -1))
    return gate, idx


def kernel(x, W1, b1, W2, b2):
    return _gating(x, W1, b1, W2, b2)


# speedup vs baseline: 6.2110x; 1.2482x over previous
import functools

import jax
import jax.numpy as jnp
from jax.experimental import pallas as pl

K = 8


def _gating_block_kernel(x_ref, w1_ref, b1_ref, w2_ref, b2_ref,
                         gate_ref, idx_ref):
    h = jnp.dot(x_ref[...], w1_ref[...], preferred_element_type=jnp.float32)
    h = jnp.maximum(h + b1_ref[...], 0.0)
    logits = jnp.dot(h, w2_ref[...], preferred_element_type=jnp.float32)
    logits = logits + b2_ref[...]

    lt = logits.T
    e, r = lt.shape
    iota = jax.lax.broadcasted_iota(jnp.int32, (e, r), 0)
    neg_inf = jnp.float32(-jnp.inf)

    work = lt
    selected = jnp.zeros((e, r), dtype=jnp.bool_)
    idx_rows = []
    top1 = None
    for k in range(K):
        m = jnp.max(work, axis=0, keepdims=True)
        if k == 0:
            top1 = m
        is_max = work == m
        idx = jnp.min(jnp.where(is_max, iota, e), axis=0, keepdims=True)
        idx_rows.append(idx)
        one_hot = iota == idx
        selected = jnp.logical_or(selected, one_hot)
        work = jnp.where(one_hot, neg_inf, work)

    p = jnp.where(selected, jnp.exp(lt - top1), 0.0)
    z = jnp.sum(p, axis=0, keepdims=True)
    gate_ref[...] = (p / z).T
    idx_ref[...] = jnp.concatenate(idx_rows, axis=0).T


@functools.partial(jax.jit, static_argnames=("block_rows",))
def _gating(x, W1, b1, W2, b2, block_rows=512):
    n, d = x.shape
    h_dim = W1.shape[1]
    e = W2.shape[1]
    grid = (n // block_rows,)
    gate, idx = pl.pallas_call(
        _gating_block_kernel,
        grid=grid,
        in_specs=[
            pl.BlockSpec((block_rows, d), lambda i: (i, 0)),
            pl.BlockSpec((d, h_dim), lambda i: (0, 0)),
            pl.BlockSpec((1, h_dim), lambda i: (0, 0)),
            pl.BlockSpec((h_dim, e), lambda i: (0, 0)),
            pl.BlockSpec((1, e), lambda i: (0, 0)),
        ],
        out_specs=[
            pl.BlockSpec((block_rows, e), lambda i: (i, 0)),
            pl.BlockSpec((block_rows, K), lambda i: (i, 0)),
        ],
        out_shape=[
            jax.ShapeDtypeStruct((n, e), jnp.float32),
            jax.ShapeDtypeStruct((n, K), jnp.int32),
        ],
    )(x, W1, b1.reshape(1, -1), W2, b2.reshape(1, -1))
    return gate, idx


def kernel(x, W1, b1, W2, b2):
    return _gating(x, W1, b1, W2, b2)


# speedup vs baseline: 6.7844x; 1.0923x over previous
import functools

import jax
import jax.numpy as jnp
from jax.experimental import pallas as pl

K = 8


def _gating_block_kernel(x_ref, w1_ref, b1_ref, w2_ref, b2_ref,
                         gate_ref, idx_ref):
    h = jnp.dot(x_ref[...], w1_ref[...], preferred_element_type=jnp.float32)
    h = jnp.maximum(h + b1_ref[...], 0.0)
    logits = jnp.dot(h, w2_ref[...], preferred_element_type=jnp.float32)
    logits = logits + b2_ref[...]

    lt = logits.T
    e, r = lt.shape
    iota = jax.lax.broadcasted_iota(jnp.int32, (e, r), 0)
    neg_inf = jnp.float32(-jnp.inf)

    work = lt
    selected = jnp.zeros((e, r), dtype=jnp.bool_)
    idx_rows = []
    top1 = None
    for k in range(K):
        m = jnp.max(work, axis=0, keepdims=True)
        if k == 0:
            top1 = m
        is_max = work == m
        idx = jnp.min(jnp.where(is_max, iota, e), axis=0, keepdims=True)
        idx_rows.append(idx)
        one_hot = iota == idx
        selected = jnp.logical_or(selected, one_hot)
        work = jnp.where(one_hot, neg_inf, work)

    p = jnp.where(selected, jnp.exp(lt - top1), 0.0)
    z = jnp.sum(p, axis=0, keepdims=True)
    gate_ref[...] = (p / z).T
    idx_ref[...] = jnp.concatenate(idx_rows, axis=0).T


@functools.partial(jax.jit, static_argnames=("block_rows",))
def _gating(x, W1, b1, W2, b2, block_rows=1024):
    n, d = x.shape
    h_dim = W1.shape[1]
    e = W2.shape[1]
    grid = (n // block_rows,)
    gate, idx = pl.pallas_call(
        _gating_block_kernel,
        grid=grid,
        in_specs=[
            pl.BlockSpec((block_rows, d), lambda i: (i, 0)),
            pl.BlockSpec((d, h_dim), lambda i: (0, 0)),
            pl.BlockSpec((1, h_dim), lambda i: (0, 0)),
            pl.BlockSpec((h_dim, e), lambda i: (0, 0)),
            pl.BlockSpec((1, e), lambda i: (0, 0)),
        ],
        out_specs=[
            pl.BlockSpec((block_rows, e), lambda i: (i, 0)),
            pl.BlockSpec((block_rows, K), lambda i: (i, 0)),
        ],
        out_shape=[
            jax.ShapeDtypeStruct((n, e), jnp.float32),
            jax.ShapeDtypeStruct((n, K), jnp.int32),
        ],
    )(x, W1, b1.reshape(1, -1), W2, b2.reshape(1, -1))
    return gate, idx


def kernel(x, W1, b1, W2, b2):
    return _gating(x, W1, b1, W2, b2)
